# baseline (speedup 1.0000x reference)
import functools

import jax
import jax.numpy as jnp
from jax import lax
from jax.experimental import pallas as pl
from jax.experimental import layout as jax_layout
from jax.experimental.pallas import tpu as pltpu
from jax.experimental.pallas import tpu_sc as plsc

NC = 2
NS = 16
L = 16

_MESH = plsc.VectorSubcoreMesh(core_axis_name="c", subcore_axis_name="s")
_SC_PARAMS = pltpu.CompilerParams(needs_layout_passes=False,
                                  use_tc_tiling_on_sc=False)


def _maxes_kernel(E, N, EPT, NPT):
    @functools.partial(
        pl.kernel,
        out_type=jax.ShapeDtypeStruct((NC * NS, 2 * L), jnp.int32),
        mesh=_MESH,
        compiler_params=_SC_PARAMS,
        scratch_types=[
            pltpu.VMEM((EPT,), jnp.int32),
            pltpu.VMEM((NPT,), jnp.int32),
            pltpu.VMEM((2 * L,), jnp.int32),
            pltpu.SemaphoreType.DMA,
            pltpu.SemaphoreType.DMA,
        ],
    )
    def k(el_hbm, nl_hbm, out_hbm, ebuf, nbuf, obuf, sem, sem2):
        c = lax.axis_index("c")
        s = lax.axis_index("s")
        wid = c * NS + s
        zero16 = jnp.zeros((L,), jnp.int32)

        d_e = pltpu.async_copy(el_hbm.at[pl.ds(wid * EPT, EPT)], ebuf, sem)
        nbase = jnp.minimum(wid * NPT, N - NPT)
        d_n = pltpu.async_copy(nl_hbm.at[pl.ds(nbase, NPT)], nbuf, sem2)
        d_n.wait()

        def ngrp(i, m2):
            return jnp.maximum(m2, nbuf[pl.ds(i * L, L)])

        mn = lax.fori_loop(0, NPT // L, ngrp, zero16)
        d_e.wait()

        def grp(i, m2):
            return jnp.maximum(m2, ebuf[pl.ds(i * L, L)])

        me = lax.fori_loop(0, EPT // L, grp, zero16)

        obuf[pl.ds(0, L)] = me
        obuf[pl.ds(L, L)] = mn
        pltpu.sync_copy(obuf, out_hbm.at[wid])

    return k


def _edge_kernel(E, N, EPT):
    B = 2000
    NBLK = EPT // B
    GP = B // L
    NR = 16
    RW = 128
    HPS = (N * L) // NS
    ZB = 2000
    TB = E // 128
    WT = 17

    @functools.partial(
        pl.kernel,
        out_type=(
            jax.ShapeDtypeStruct((E,), jnp.float32),
            jax.ShapeDtypeStruct((E,), jnp.float32),
            jax.ShapeDtypeStruct((E,), jnp.float32),
            jax.ShapeDtypeStruct((NC, N * L), jnp.float32),
        ),
        mesh=_MESH,
        compiler_params=_SC_PARAMS,
        scratch_types=[
            pltpu.VMEM((N,), jnp.int32),
            pltpu.VMEM((WT, 2, 128), jnp.int32),
            pltpu.VMEM((B,), jnp.int32),
            pltpu.VMEM((B,), jnp.float32),
            pltpu.VMEM((B,), jnp.float32),
            pltpu.VMEM((B,), jnp.float32),
            pltpu.VMEM((NR, RW), jnp.int32),
            pltpu.VMEM((NR, RW), jnp.float32),
            pltpu.VMEM((ZB,), jnp.float32),
            pltpu.VMEM((2, L), jnp.float32),
            pltpu.VMEM_SHARED((N * L,), jnp.float32),
            pltpu.SemaphoreType.DMA,
            pltpu.SemaphoreType.DMA,
            pltpu.SemaphoreType.DMA,
        ],
    )
    def k(ei_hbm, el_hbm, nl_hbm, inv_hbm, c0_hbm, c1_hbm, c2_hbm, part_hbm,
          nl_v, ei_v, elab_v, c0_v, c1_v, c2_v, hidx_v, ones_v, zb_v, inv_v,
          hist_sh, sem_in, sem_sc, sem_out):
        c = lax.axis_index("c")
        s = lax.axis_index("s")
        wid = c * NS + s
        iota16 = lax.iota(jnp.int32, L)
        fz16 = jnp.zeros((L,), jnp.float32)
        fo16 = jnp.ones((L,), jnp.float32)

        pltpu.sync_copy(nl_hbm, nl_v)
        pltpu.sync_copy(inv_hbm, inv_v)

        def z_hidx(i, _):
            hidx_v[i // 8, pl.ds((i % 8) * L, L)] = jnp.zeros((L,), jnp.int32)
            ones_v[i // 8, pl.ds((i % 8) * L, L)] = jnp.where(
                i * L + iota16 < B, 1.0, 0.0)
            return 0

        lax.fori_loop(0, NR * 8, z_hidx, 0)

        def z_zb(i, _):
            zb_v[pl.ds(i * L, L)] = fz16
            return 0

        lax.fori_loop(0, ZB // L, z_zb, 0)

        def z_hist(j, _):
            pltpu.sync_copy(zb_v, hist_sh.at[pl.ds(s * HPS + j * ZB, ZB)])
            return 0

        lax.fori_loop(0, HPS // ZB, z_hist, 0)
        plsc.subcore_barrier()

        def blk(b, _):
            base = wid * EPT + b * B
            tile_lo = jnp.minimum(base // 128, TB - WT)
            off = base - tile_lo * 128
            d_ei = pltpu.async_copy(ei_hbm.at[pl.ds(tile_lo, WT)], ei_v,
                                    sem_in)
            d_el = pltpu.async_copy(el_hbm.at[pl.ds(base, B)], elab_v, sem_in)
            d_ei.wait()
            d_el.wait()

            descs = []
            for sb in range(NR):
                ngr = min(8, GP - sb * 8)

                def grp(i, _, sb=sb):
                    g = sb * 8 + i
                    sl = pl.ds(g * L, L)
                    goff = off + g * L
                    t = goff // 128
                    cidx = goff - t * 128
                    si = ei_v[t, 0, pl.ds(cidx, L)]
                    di = ei_v[t, 1, pl.ds(cidx, L)]
                    el = elab_v[sl]
                    invmn = inv_v[0, :]
                    invme = inv_v[1, :]
                    u = plsc.load_gather(nl_v, [si]) + 1
                    v = plsc.load_gather(nl_v, [di]) + 1
                    uf = u.astype(jnp.float32) * invmn
                    vf = v.astype(jnp.float32) * invmn
                    wf = (el + 1).astype(jnp.float32) * invme
                    r = 1.0 / (uf + vf + wf)
                    c0_v[sl] = uf * r
                    c1_v[sl] = vf * r
                    c2_v[sl] = wf * r
                    hidx_v[sb, pl.ds(i * L, L)] = di * L + el
                    return 0

                lax.fori_loop(0, ngr, grp, 0)
                descs.append(pltpu.async_copy(
                    ones_v.at[sb], hist_sh.at[hidx_v.at[sb]], sem_sc,
                    add=True))

            descs.append(pltpu.async_copy(c0_v, c0_hbm.at[pl.ds(base, B)],
                                          sem_out))
            descs.append(pltpu.async_copy(c1_v, c1_hbm.at[pl.ds(base, B)],
                                          sem_out))
            descs.append(pltpu.async_copy(c2_v, c2_hbm.at[pl.ds(base, B)],
                                          sem_out))
            for d in descs:
                d.wait()
            return 0

        lax.fori_loop(0, NBLK, blk, 0)
        plsc.subcore_barrier()
        pltpu.sync_copy(hist_sh.at[pl.ds(s * HPS, HPS)],
                        part_hbm.at[c, pl.ds(s * HPS, HPS)])

    return k


def _nfeat_kernel(N, D):
    BR = 400
    grid = N // BR

    def body(lab_ref, part_ref, out_ref):
        lab = lab_ref[...]
        cols = lax.broadcasted_iota(jnp.int32, (BR, D), 1)
        oh = jnp.where(cols == lab, 1.0, 0.0).astype(jnp.float32)
        cnt = part_ref[0] + part_ref[1]
        deg = jnp.sum(cnt, axis=1, keepdims=True)
        emb = cnt / jnp.maximum(deg, 1.0)
        out_ref[:, :D] = oh
        out_ref[:, D:] = emb

    return pl.pallas_call(
        body,
        grid=(grid,),
        in_specs=[
            pl.BlockSpec((BR, 1), lambda i: (i, 0)),
            pl.BlockSpec((NC, BR, L), lambda i: (0, i, 0)),
        ],
        out_specs=pl.BlockSpec((BR, D + L), lambda i: (i, 0)),
        out_shape=jax.ShapeDtypeStruct((N, D + L), jnp.float32),
    )


def kernel(node_labels, edge_labels, edge_index, node_encoder, edge_encoder):
    N = node_labels.shape[0]
    E = edge_labels.shape[0]
    D = node_encoder.shape[0]
    NW = NC * NS
    EPT = E // NW
    assert E == EPT * NW and EPT % 2000 == 0
    NPT = 1568
    assert NPT % L == 0 and (N * L) % (NS * 2000) == 0

    node_labels = node_labels.astype(jnp.int32)
    edge_labels = edge_labels.astype(jnp.int32)
    edge_index = edge_index.astype(jnp.int32)

    mx = _maxes_kernel(E, N, EPT, NPT)(edge_labels, node_labels)
    max_el = jnp.max(mx[:, :L]).astype(jnp.float32)
    max_nl = (jnp.max(mx[:, L:]) + 1).astype(jnp.float32)
    inv = jnp.stack([jnp.full((L,), 1.0, jnp.float32) / max_nl,
                     jnp.full((L,), 1.0, jnp.float32) / max_el])

    ei_tiles = edge_index.reshape(2, E // 128, 128).transpose(1, 0, 2)
    c0, c1, c2, parts = _edge_kernel(E, N, EPT)(
        ei_tiles, edge_labels, node_labels, inv)
    e_feat = jnp.stack([c0, c1, c2], axis=1)

    n_feat = _nfeat_kernel(N, D)(
        node_labels.reshape(N, 1), parts.reshape(NC, N, L))
    return n_feat, e_feat


# speedup vs baseline: 1.0504x; 1.0504x over previous
import functools

import jax
import jax.numpy as jnp
from jax import lax
from jax.experimental import pallas as pl
from jax.experimental import layout as jax_layout
from jax.experimental.pallas import tpu as pltpu
from jax.experimental.pallas import tpu_sc as plsc

NC = 2
NS = 16
L = 16

_MESH = plsc.VectorSubcoreMesh(core_axis_name="c", subcore_axis_name="s")
_SC_PARAMS = pltpu.CompilerParams(needs_layout_passes=False,
                                  use_tc_tiling_on_sc=False)


def _maxes_kernel(E, N, EPT, NPT):
    @functools.partial(
        pl.kernel,
        out_type=jax.ShapeDtypeStruct((NC * NS, 2 * L), jnp.int32),
        mesh=_MESH,
        compiler_params=_SC_PARAMS,
        scratch_types=[
            pltpu.VMEM((EPT,), jnp.int32),
            pltpu.VMEM((NPT,), jnp.int32),
            pltpu.VMEM((2 * L,), jnp.int32),
            pltpu.SemaphoreType.DMA,
            pltpu.SemaphoreType.DMA,
        ],
    )
    def k(el_hbm, nl_hbm, out_hbm, ebuf, nbuf, obuf, sem, sem2):
        c = lax.axis_index("c")
        s = lax.axis_index("s")
        wid = c * NS + s
        zero16 = jnp.zeros((L,), jnp.int32)

        d_e = pltpu.async_copy(el_hbm.at[pl.ds(wid * EPT, EPT)], ebuf, sem)
        nbase = jnp.minimum(wid * NPT, N - NPT)
        d_n = pltpu.async_copy(nl_hbm.at[pl.ds(nbase, NPT)], nbuf, sem2)
        d_n.wait()

        def ngrp(i, m2):
            return jnp.maximum(m2, nbuf[pl.ds(i * L, L)])

        mn = lax.fori_loop(0, NPT // L, ngrp, zero16)
        d_e.wait()

        def grp(i, m2):
            return jnp.maximum(m2, ebuf[pl.ds(i * L, L)])

        me = lax.fori_loop(0, EPT // L, grp, zero16)

        obuf[pl.ds(0, L)] = me
        obuf[pl.ds(L, L)] = mn
        pltpu.sync_copy(obuf, out_hbm.at[wid])

    return k


def _edge_kernel(E, N, EPT):
    B = 2000
    NBLK = EPT // B
    GP = B // L
    NR = 16
    RW = 128
    HPS = (N * L) // NS
    ZB = 2000
    TB = E // 128
    WT = 17

    @functools.partial(
        pl.kernel,
        out_type=(
            jax.ShapeDtypeStruct((E,), jnp.float32),
            jax.ShapeDtypeStruct((E,), jnp.float32),
            jax.ShapeDtypeStruct((E,), jnp.float32),
            jax.ShapeDtypeStruct((NC, N * L), jnp.float32),
        ),
        mesh=_MESH,
        compiler_params=_SC_PARAMS,
        scratch_types=[
            pltpu.VMEM((N,), jnp.int32),
            pltpu.VMEM((2, WT, 2, 128), jnp.int32),
            pltpu.VMEM((2, B), jnp.int32),
            pltpu.VMEM((B,), jnp.float32),
            pltpu.VMEM((B,), jnp.float32),
            pltpu.VMEM((B,), jnp.float32),
            pltpu.VMEM((NR, RW), jnp.int32),
            pltpu.VMEM((NR, RW), jnp.float32),
            pltpu.VMEM((2, L), jnp.float32),
            pltpu.VMEM_SHARED((N * L,), jnp.float32),
            pltpu.SemaphoreType.DMA,
            pltpu.SemaphoreType.DMA,
            pltpu.SemaphoreType.DMA,
        ],
    )
    def k(ei_hbm, el_hbm, nl_hbm, inv_hbm, c0_hbm, c1_hbm, c2_hbm, part_hbm,
          nl_v, ei_v, elab_v, c0_v, c1_v, c2_v, hidx_v, ones_v, inv_v,
          hist_sh, sem_in, sem_sc, sem_out):
        c = lax.axis_index("c")
        s = lax.axis_index("s")
        wid = c * NS + s
        iota16 = lax.iota(jnp.int32, L)
        fz16 = jnp.zeros((L,), jnp.float32)
        fo16 = jnp.ones((L,), jnp.float32)

        pltpu.sync_copy(nl_hbm, nl_v)
        pltpu.sync_copy(inv_hbm, inv_v)

        def z_hidx(i, _):
            hidx_v[i // 8, pl.ds((i % 8) * L, L)] = jnp.zeros((L,), jnp.int32)
            ones_v[i // 8, pl.ds((i % 8) * L, L)] = jnp.where(
                i * L + iota16 < B, 1.0, 0.0)
            return 0

        lax.fori_loop(0, NR * 8, z_hidx, 0)

        def z_zb(i, _):
            c0_v[pl.ds(i * L, L)] = fz16
            return 0

        lax.fori_loop(0, ZB // L, z_zb, 0)

        def z_hist(j, _):
            pltpu.sync_copy(c0_v, hist_sh.at[pl.ds(s * HPS + j * ZB, ZB)])
            return 0

        lax.fori_loop(0, HPS // ZB, z_hist, 0)

        def lo(b, slot):
            base0 = wid * EPT + b * B
            tlo = jnp.minimum(base0 // 128, TB - WT)
            pltpu.async_copy(ei_hbm.at[pl.ds(tlo, WT)], ei_v.at[slot], sem_in)
            pltpu.async_copy(el_hbm.at[pl.ds(base0, B)], elab_v.at[slot],
                             sem_in)

        lo(0, 0)
        plsc.subcore_barrier()

        def blk(b, _):
            q = lax.rem(b, 2)
            base = wid * EPT + b * B
            tile_lo = jnp.minimum(base // 128, TB - WT)
            off = base - tile_lo * 128
            pltpu.make_async_copy(ei_hbm.at[pl.ds(0, WT)], ei_v.at[q],
                                  sem_in).wait()
            pltpu.make_async_copy(el_hbm.at[pl.ds(0, B)], elab_v.at[q],
                                  sem_in).wait()

            @pl.when(b + 1 < NBLK)
            def _prefetch():
                lo(b + 1, 1 - q)

            descs = []
            for sb in range(NR):
                ngr = min(8, GP - sb * 8)

                def grp(i, _, sb=sb):
                    g = sb * 8 + i
                    sl = pl.ds(g * L, L)
                    goff = off + g * L
                    t = goff // 128
                    cidx = goff - t * 128
                    si = ei_v[q, t, 0, pl.ds(cidx, L)]
                    di = ei_v[q, t, 1, pl.ds(cidx, L)]
                    el = elab_v[q, sl]
                    invmn = inv_v[0, :]
                    invme = inv_v[1, :]
                    u = plsc.load_gather(nl_v, [si]) + 1
                    v = plsc.load_gather(nl_v, [di]) + 1
                    uf = u.astype(jnp.float32) * invmn
                    vf = v.astype(jnp.float32) * invmn
                    wf = (el + 1).astype(jnp.float32) * invme
                    r = 1.0 / (uf + vf + wf)
                    c0_v[sl] = uf * r
                    c1_v[sl] = vf * r
                    c2_v[sl] = wf * r
                    hidx_v[sb, pl.ds(i * L, L)] = di * L + el
                    return 0

                lax.fori_loop(0, ngr, grp, 0)
                descs.append(pltpu.async_copy(
                    ones_v.at[sb], hist_sh.at[hidx_v.at[sb]], sem_sc,
                    add=True))

            descs.append(pltpu.async_copy(c0_v, c0_hbm.at[pl.ds(base, B)],
                                          sem_out))
            descs.append(pltpu.async_copy(c1_v, c1_hbm.at[pl.ds(base, B)],
                                          sem_out))
            descs.append(pltpu.async_copy(c2_v, c2_hbm.at[pl.ds(base, B)],
                                          sem_out))
            for d in descs:
                d.wait()
            return 0

        lax.fori_loop(0, NBLK, blk, 0)
        plsc.subcore_barrier()
        pltpu.sync_copy(hist_sh.at[pl.ds(s * HPS, HPS)],
                        part_hbm.at[c, pl.ds(s * HPS, HPS)])

    return k


def _nfeat_kernel(N, D):
    BR = 400
    grid = N // BR

    def body(lab_ref, part_ref, out_ref):
        lab = lab_ref[...]
        cols = lax.broadcasted_iota(jnp.int32, (BR, D), 1)
        oh = jnp.where(cols == lab, 1.0, 0.0).astype(jnp.float32)
        cnt = part_ref[0] + part_ref[1]
        deg = jnp.sum(cnt, axis=1, keepdims=True)
        emb = cnt / jnp.maximum(deg, 1.0)
        out_ref[:, :D] = oh
        out_ref[:, D:] = emb

    return pl.pallas_call(
        body,
        grid=(grid,),
        in_specs=[
            pl.BlockSpec((BR, 1), lambda i: (i, 0)),
            pl.BlockSpec((NC, BR, L), lambda i: (0, i, 0)),
        ],
        out_specs=pl.BlockSpec((BR, D + L), lambda i: (i, 0)),
        out_shape=jax.ShapeDtypeStruct((N, D + L), jnp.float32),
    )


def kernel(node_labels, edge_labels, edge_index, node_encoder, edge_encoder):
    N = node_labels.shape[0]
    E = edge_labels.shape[0]
    D = node_encoder.shape[0]
    NW = NC * NS
    EPT = E // NW
    assert E == EPT * NW and EPT % 2000 == 0
    NPT = 1568
    assert NPT % L == 0 and (N * L) % (NS * 2000) == 0

    node_labels = node_labels.astype(jnp.int32)
    edge_labels = edge_labels.astype(jnp.int32)
    edge_index = edge_index.astype(jnp.int32)

    mx = _maxes_kernel(E, N, EPT, NPT)(edge_labels, node_labels)
    max_el = jnp.max(mx[:, :L]).astype(jnp.float32)
    max_nl = (jnp.max(mx[:, L:]) + 1).astype(jnp.float32)
    inv = jnp.stack([jnp.full((L,), 1.0, jnp.float32) / max_nl,
                     jnp.full((L,), 1.0, jnp.float32) / max_el])

    ei_tiles = edge_index.reshape(2, E // 128, 128).transpose(1, 0, 2)
    c0, c1, c2, parts = _edge_kernel(E, N, EPT)(
        ei_tiles, edge_labels, node_labels, inv)
    e_feat = jnp.stack([c0, c1, c2], axis=1)

    n_feat = _nfeat_kernel(N, D)(
        node_labels.reshape(N, 1), parts.reshape(NC, N, L))
    return n_feat, e_feat


# speedup vs baseline: 1.0512x; 1.0008x over previous
import functools

import jax
import jax.numpy as jnp
from jax import lax
from jax.experimental import pallas as pl
from jax.experimental.pallas import tpu as pltpu
from jax.experimental.pallas import tpu_sc as plsc

NC = 2
NS = 16
L = 16

_MESH = plsc.VectorSubcoreMesh(core_axis_name="c", subcore_axis_name="s")
_SC_PARAMS = pltpu.CompilerParams(needs_layout_passes=False,
                                  use_tc_tiling_on_sc=False)


def _maxes_kernel(E, N, EPT, NPT):
    @functools.partial(
        pl.kernel,
        out_type=jax.ShapeDtypeStruct((NC * NS, 2 * L), jnp.int32),
        mesh=_MESH,
        compiler_params=_SC_PARAMS,
        scratch_types=[
            pltpu.VMEM((EPT,), jnp.int32),
            pltpu.VMEM((NPT,), jnp.int32),
            pltpu.VMEM((2 * L,), jnp.int32),
            pltpu.SemaphoreType.DMA,
            pltpu.SemaphoreType.DMA,
        ],
    )
    def k(el_hbm, nl_hbm, out_hbm, ebuf, nbuf, obuf, sem, sem2):
        c = lax.axis_index("c")
        s = lax.axis_index("s")
        wid = c * NS + s
        zero16 = jnp.zeros((L,), jnp.int32)

        d_e = pltpu.async_copy(el_hbm.at[pl.ds(wid * EPT, EPT)], ebuf, sem)
        nbase = jnp.minimum(wid * NPT, N - NPT)
        d_n = pltpu.async_copy(nl_hbm.at[pl.ds(nbase, NPT)], nbuf, sem2)
        d_n.wait()

        def ngrp(i, m2):
            return jnp.maximum(m2, nbuf[pl.ds(i * L, L)])

        mn = lax.fori_loop(0, NPT // L, ngrp, zero16)
        d_e.wait()

        def grp(i, m2):
            return jnp.maximum(m2, ebuf[pl.ds(i * L, L)])

        me = lax.fori_loop(0, EPT // L, grp, zero16)

        obuf[pl.ds(0, L)] = me
        obuf[pl.ds(L, L)] = mn
        pltpu.sync_copy(obuf, out_hbm.at[wid])

    return k


def _edge_kernel(E, N, EPT):
    B = 2000
    NBLK = EPT // B
    GP = B // L
    NR = 16
    RW = 128
    HPS = (N * L) // NS
    ZB = 2000
    TB = E // 128
    WT = 17

    @functools.partial(
        pl.kernel,
        out_type=(
            jax.ShapeDtypeStruct((E,), jnp.float32),
            jax.ShapeDtypeStruct((E,), jnp.float32),
            jax.ShapeDtypeStruct((E,), jnp.float32),
            jax.ShapeDtypeStruct((NC, N * L), jnp.float32),
        ),
        mesh=_MESH,
        compiler_params=_SC_PARAMS,
        scratch_types=[
            pltpu.VMEM((N,), jnp.int32),
            pltpu.VMEM((2, WT, 2, 128), jnp.int32),
            pltpu.VMEM((2, B), jnp.int32),
            pltpu.VMEM((B,), jnp.float32),
            pltpu.VMEM((B,), jnp.float32),
            pltpu.VMEM((B,), jnp.float32),
            pltpu.VMEM((NR, RW), jnp.int32),
            pltpu.VMEM((NR, RW), jnp.float32),
            pltpu.VMEM((2, L), jnp.float32),
            pltpu.VMEM_SHARED((N * L,), jnp.float32),
            pltpu.SemaphoreType.DMA,
            pltpu.SemaphoreType.DMA,
            pltpu.SemaphoreType.DMA,
        ],
    )
    def k(ei_hbm, el_hbm, nl_hbm, inv_hbm, c0_hbm, c1_hbm, c2_hbm, part_hbm,
          nl_v, ei_v, elab_v, c0_v, c1_v, c2_v, hidx_v, ones_v, inv_v,
          hist_sh, sem_in, sem_sc, sem_out):
        c = lax.axis_index("c")
        s = lax.axis_index("s")
        wid = c * NS + s
        iota16 = lax.iota(jnp.int32, L)
        fz16 = jnp.zeros((L,), jnp.float32)

        pltpu.sync_copy(nl_hbm, nl_v)
        pltpu.sync_copy(inv_hbm, inv_v)

        def z_hidx(i, _):
            hidx_v[i // 8, pl.ds((i % 8) * L, L)] = jnp.zeros((L,), jnp.int32)
            ones_v[i // 8, pl.ds((i % 8) * L, L)] = jnp.where(
                i * L + iota16 < B, 1.0, 0.0)
            return 0

        lax.fori_loop(0, NR * 8, z_hidx, 0)

        def z_zb(i, _):
            c0_v[pl.ds(i * L, L)] = fz16
            return 0

        lax.fori_loop(0, ZB // L, z_zb, 0)

        def z_hist(j, _):
            pltpu.sync_copy(c0_v, hist_sh.at[pl.ds(s * HPS + j * ZB, ZB)])
            return 0

        lax.fori_loop(0, HPS // ZB, z_hist, 0)

        def lo(b, slot):
            base0 = wid * EPT + b * B
            tlo = jnp.minimum(base0 // 128, TB - WT)
            pltpu.async_copy(ei_hbm.at[pl.ds(tlo, WT)], ei_v.at[slot], sem_in)
            pltpu.async_copy(el_hbm.at[pl.ds(base0, B)], elab_v.at[slot],
                             sem_in)

        lo(0, 0)
        plsc.subcore_barrier()

        def blk(b, _):
            q = lax.rem(b, 2)
            base = wid * EPT + b * B
            tile_lo = jnp.minimum(base // 128, TB - WT)
            off = base - tile_lo * 128
            pltpu.make_async_copy(ei_hbm.at[pl.ds(0, WT)], ei_v.at[q],
                                  sem_in).wait()
            pltpu.make_async_copy(el_hbm.at[pl.ds(0, B)], elab_v.at[q],
                                  sem_in).wait()

            @pl.when(b + 1 < NBLK)
            def _prefetch():
                lo(b + 1, 1 - q)

            descs = []
            for sb in range(NR):
                ngr = min(8, GP - sb * 8)

                def grp(i, _, sb=sb):
                    g = sb * 8 + i
                    sl = pl.ds(g * L, L)
                    goff = off + g * L
                    t = goff // 128
                    cidx = goff - t * 128
                    si = ei_v[q, t, 0, pl.ds(cidx, L)]
                    di = ei_v[q, t, 1, pl.ds(cidx, L)]
                    el = elab_v[q, sl]
                    invmn = inv_v[0, :]
                    invme = inv_v[1, :]
                    u = plsc.load_gather(nl_v, [si]) + 1
                    v = plsc.load_gather(nl_v, [di]) + 1
                    uf = u.astype(jnp.float32) * invmn
                    vf = v.astype(jnp.float32) * invmn
                    wf = (el + 1).astype(jnp.float32) * invme
                    r = 1.0 / (uf + vf + wf)
                    c0_v[sl] = uf * r
                    c1_v[sl] = vf * r
                    c2_v[sl] = wf * r
                    hidx_v[sb, pl.ds(i * L, L)] = di * L + el
                    return 0

                lax.fori_loop(0, ngr, grp, 0)
                descs.append(pltpu.async_copy(
                    ones_v.at[sb], hist_sh.at[hidx_v.at[sb]], sem_sc,
                    add=True))

            descs.append(pltpu.async_copy(c0_v, c0_hbm.at[pl.ds(base, B)],
                                          sem_out))
            descs.append(pltpu.async_copy(c1_v, c1_hbm.at[pl.ds(base, B)],
                                          sem_out))
            descs.append(pltpu.async_copy(c2_v, c2_hbm.at[pl.ds(base, B)],
                                          sem_out))
            for d in descs:
                d.wait()
            return 0

        lax.fori_loop(0, NBLK, blk, 0)
        plsc.subcore_barrier()
        pltpu.sync_copy(hist_sh.at[pl.ds(s * HPS, HPS)],
                        part_hbm.at[c, pl.ds(s * HPS, HPS)])

    return k


def _nfeat_kernel(N, D):
    BR = 400
    grid = N // BR

    def body(lab_ref, part_ref, out_ref):
        lab = lab_ref[...]
        cols = lax.broadcasted_iota(jnp.int32, (BR, D), 1)
        oh = jnp.where(cols == lab, 1.0, 0.0).astype(jnp.float32)
        cnt = part_ref[0] + part_ref[1]
        deg = jnp.sum(cnt, axis=1, keepdims=True)
        emb = cnt / jnp.maximum(deg, 1.0)
        out_ref[:, :D] = oh
        out_ref[:, D:] = emb

    return pl.pallas_call(
        body,
        grid=(grid,),
        in_specs=[
            pl.BlockSpec((BR, 1), lambda i: (i, 0)),
            pl.BlockSpec((NC, BR, L), lambda i: (0, i, 0)),
        ],
        out_specs=pl.BlockSpec((BR, D + L), lambda i: (i, 0)),
        out_shape=jax.ShapeDtypeStruct((N, D + L), jnp.float32),
    )


def kernel(node_labels, edge_labels, edge_index, node_encoder, edge_encoder):
    N = node_labels.shape[0]
    E = edge_labels.shape[0]
    D = node_encoder.shape[0]
    NW = NC * NS
    EPT = E // NW
    assert E == EPT * NW and EPT % 2000 == 0
    NPT = 1568
    assert NPT % L == 0 and (N * L) % (NS * 2000) == 0

    node_labels = node_labels.astype(jnp.int32)
    edge_labels = edge_labels.astype(jnp.int32)
    edge_index = edge_index.astype(jnp.int32)

    mx = _maxes_kernel(E, N, EPT, NPT)(edge_labels, node_labels)
    max_el = jnp.max(mx[:, :L]).astype(jnp.float32)
    max_nl = (jnp.max(mx[:, L:]) + 1).astype(jnp.float32)
    inv = jnp.stack([jnp.full((L,), 1.0, jnp.float32) / max_nl,
                     jnp.full((L,), 1.0, jnp.float32) / max_el])

    ei_tiles = edge_index.reshape(2, E // 128, 128).transpose(1, 0, 2)
    c0, c1, c2, parts = _edge_kernel(E, N, EPT)(
        ei_tiles, edge_labels, node_labels, inv)
    e_feat = jnp.stack([c0, c1, c2], axis=1)

    n_feat = _nfeat_kernel(N, D)(
        node_labels.reshape(N, 1), parts.reshape(NC, N, L))
    return n_feat, e_feat
